# baseline (speedup 1.0000x reference)
import functools

import jax
import jax.numpy as jnp
import numpy as np
from jax import lax
from jax.experimental import pallas as pl
from jax.experimental.pallas import tpu as pltpu
from jax.experimental.pallas import tpu_sc as plsc

_IDX_LO = np.concatenate([np.arange(16) + 32 * u for u in range(4)])
_IDX_HI = _IDX_LO + 16

N = 10000
E = 320000
D_NODE = 128
D_EDGE = 16
H = 128
OUT = 128
L = 3
HW = H // 2

NC = 2
NS = 16
NW = NC * NS
EPW = E // NW
CH = 40
NCHUNK = EPW // CH
NPAD = 10240
ROWS_PER_TILE = NPAD // NS
ZCHUNK = ROWS_PER_TILE // CH


def _sc_message_pass(h_hbm, e_hbm, src_hbm, dst_hbm, out_hbm,
                     table, sidx, didx,
                     hbuf0, hbuf1, ebuf0, ebuf1, mbuf,
                     sem_e0, sem_e1, sem_g0, sem_g1, sem_s,
                     sem_i0, sem_i1):
    c = lax.axis_index("c")
    s = lax.axis_index("s")
    wid = s * NC + c
    hb = (hbuf0, hbuf1)
    eb = (ebuf0, ebuf1)
    se = (sem_e0, sem_e1)
    sg = (sem_g0, sem_g1)
    si = (sem_i0, sem_i1)

    zero = jnp.zeros((16,), jnp.float32)

    def _zrow(i, carry):
        for g in range(8):
            mbuf[i, pl.ds(g * 16, 16)] = zero
        return carry

    lax.fori_loop(0, CH, _zrow, 0)
    for k in range(ZCHUNK):
        pltpu.sync_copy(mbuf, table.at[pl.ds(s * ROWS_PER_TILE + k * CH, CH)])
    plsc.subcore_barrier()

    def _issue_idx(k, p):
        base = wid * EPW + k * CH
        pltpu.async_copy(src_hbm.at[pl.ds(base, CH)], sidx.at[k % 8, 0], si[p])
        pltpu.async_copy(dst_hbm.at[pl.ds(base, CH)], didx.at[k % 8, 0], si[p])

    def _wait_idx(k, p):
        base = wid * EPW + k * CH
        pltpu.make_async_copy(src_hbm.at[pl.ds(base, CH)],
                              sidx.at[k % 8, 0], si[p]).wait()
        pltpu.make_async_copy(dst_hbm.at[pl.ds(base, CH)],
                              didx.at[k % 8, 0], si[p]).wait()

    def _issue_ge(k, b):
        pltpu.async_copy(e_hbm.at[pl.ds(wid * EPW + k * CH, CH)], eb[b], se[b])
        pltpu.async_copy(h_hbm.at[sidx.at[k % 8, 0]], hb[b], sg[b])

    def _wait_in(k, b):
        pltpu.make_async_copy(e_hbm.at[pl.ds(wid * EPW + k * CH, CH)],
                              eb[b], se[b]).wait()
        pltpu.make_async_copy(h_hbm.at[sidx.at[k % 8, 0]], hb[b], sg[b]).wait()

    hi_mask = jnp.int32(-65536)

    def _compute(b):
        hr, er, mr = hb[b], eb[b], mbuf

        def _row(i, cc):
            for u in range(4):
                we = er[i, pl.ds(u * 16, 16)]
                elo = lax.bitcast_convert_type(we << 16, jnp.float32)
                ehi = lax.bitcast_convert_type(we & hi_mask, jnp.float32)
                h0 = hr[i, pl.ds(u * 32, 16)]
                h1 = hr[i, pl.ds(u * 32 + 16, 16)]
                mr[i, pl.ds(u * 32, 16)] = jnp.maximum(h0 + elo, 0.0)
                mr[i, pl.ds(u * 32 + 16, 16)] = jnp.maximum(h1 + ehi, 0.0)
            return cc

        lax.fori_loop(0, CH, _row, 0)

    def _issue_sc(k):
        pltpu.async_copy(mbuf, table.at[didx.at[k % 8, 0]], sem_s, add=True)

    def _wait_sc(k):
        pltpu.make_async_copy(mbuf, table.at[didx.at[k % 8, 0]], sem_s).wait()

    def _step(k, b, wait_sc=True, ge2=True, idx4=True):
        _wait_in(k, b)
        if wait_sc:
            _wait_sc(k - 1)
        _compute(b)
        _issue_sc(k)
        if ge2:
            _wait_idx(k + 2, b)
            _issue_ge(k + 2, b)
        if idx4:
            _issue_idx(k + 4, b)

    _issue_idx(0, 0)
    _issue_idx(1, 1)
    _wait_idx(0, 0)
    _issue_ge(0, 0)
    _issue_idx(2, 0)
    _wait_idx(1, 1)
    _issue_ge(1, 1)
    _issue_idx(3, 1)
    _step(0, 0, wait_sc=False)
    _step(1, 1)

    def _pair(j, carry):
        for b in range(2):
            _step(2 * j + b, b)
        return carry

    _tail0 = 2 * ((NCHUNK - 4) // 2)
    lax.fori_loop(1, _tail0 // 2, _pair, 0)
    for k in range(_tail0, NCHUNK):
        _step(k, k % 2, ge2=(k + 2 < NCHUNK), idx4=(k + 4 < NCHUNK))
    _wait_sc(NCHUNK - 1)

    plsc.subcore_barrier()
    pltpu.sync_copy(table.at[pl.ds(s * ROWS_PER_TILE, ROWS_PER_TILE)],
                    out_hbm.at[c, pl.ds(s * ROWS_PER_TILE, ROWS_PER_TILE)])


@functools.cache
def _get_sc_kernel():
    return pl.kernel(
        _sc_message_pass,
        out_type=jax.ShapeDtypeStruct((NC, NPAD, H), jnp.float32),
        mesh=plsc.VectorSubcoreMesh(core_axis_name="c", subcore_axis_name="s",
                                    num_cores=NC, num_subcores=NS),
        scratch_types=[
            pltpu.VMEM_SHARED((NPAD, H), jnp.float32),
            pltpu.VMEM((8, 1, CH), jnp.int32),
            pltpu.VMEM((8, 1, CH), jnp.int32),
            pltpu.VMEM((CH, H), jnp.float32),
            pltpu.VMEM((CH, H), jnp.float32),
            pltpu.VMEM((CH, HW), jnp.int32),
            pltpu.VMEM((CH, HW), jnp.int32),
            pltpu.VMEM((CH, H), jnp.float32),
            pltpu.SemaphoreType.DMA,
            pltpu.SemaphoreType.DMA,
            pltpu.SemaphoreType.DMA,
            pltpu.SemaphoreType.DMA,
            pltpu.SemaphoreType.DMA,
            pltpu.SemaphoreType.DMA,
            pltpu.SemaphoreType.DMA,
        ],
    )


def _sc_kernel(h, e_l, src, dst):
    return _get_sc_kernel()(h, e_l, src, dst)


def _pack_words(lo, hi):
    lo_u = jax.lax.bitcast_convert_type(lo.astype(jnp.bfloat16),
                                        jnp.uint16).astype(jnp.uint32)
    hi_u = jax.lax.bitcast_convert_type(hi.astype(jnp.bfloat16),
                                        jnp.uint16).astype(jnp.uint32)
    return jax.lax.bitcast_convert_type(lo_u | (hi_u << 16), jnp.int32)


def _node_encode_body(x_ref, w_ref, b_ref, o_ref):
    o_ref[...] = jnp.dot(x_ref[...], w_ref[...],
                         preferred_element_type=jnp.float32) + b_ref[...]


def _edge_body(ea_ref, we_ref, be_ref, wl_ref, wh_ref, bl_ref, bh_ref, o_ref):
    ea = ea_ref[...]
    wcl = jnp.dot(we_ref[...], wl_ref[...], preferred_element_type=jnp.float32)
    wch = jnp.dot(we_ref[...], wh_ref[...], preferred_element_type=jnp.float32)
    bcl = jnp.dot(be_ref[...], wl_ref[...],
                  preferred_element_type=jnp.float32) + bl_ref[...]
    bch = jnp.dot(be_ref[...], wh_ref[...],
                  preferred_element_type=jnp.float32) + bh_ref[...]
    lo = jnp.dot(ea, wcl, preferred_element_type=jnp.float32) + bcl
    hi = jnp.dot(ea, wch, preferred_element_type=jnp.float32) + bch
    o_ref[...] = _pack_words(lo, hi)


def _update_body(h_ref, p_ref, w_ref, b_ref, o_ref):
    hp = h_ref[...] + p_ref[0] + p_ref[1]
    y = jnp.dot(hp, w_ref[...], preferred_element_type=jnp.float32) + b_ref[...]
    o_ref[...] = jnp.where(y >= 0, y, 0.01 * y)


def _final_body(h_ref, p_ref, skip_ref, wnn_ref, bnn_ref, wh_ref, bh_ref,
                g_ref, bb_ref, o_ref):
    hp = h_ref[...] + p_ref[0] + p_ref[1]
    y = jnp.dot(hp, wnn_ref[...], preferred_element_type=jnp.float32) + bnn_ref[...]
    h3 = jnp.where(y >= 0, y, 0.01 * y)
    hf = skip_ref[...] + h3
    out = jnp.dot(hf, wh_ref[...], preferred_element_type=jnp.float32) + bh_ref[...]
    mu = jnp.mean(out, axis=-1, keepdims=True)
    var = jnp.mean((out - mu) ** 2, axis=-1, keepdims=True)
    o_ref[...] = (out - mu) * lax.rsqrt(var + 1e-5) * g_ref[...] + bb_ref[...]


_BN = 2000
_BE = 4000


def _full(shape):
    return pl.BlockSpec(shape, lambda i: tuple(0 for _ in shape))


def kernel(x, edge_attr, edge_index, W_node, b_node, W_edge, b_edge,
           W_elin, b_elin, W_nn, b_nn, W_head, b_head, ln_g, ln_b):
    src = edge_index[0]
    dst = edge_index[1]
    b_node2 = b_node.reshape(1, H)
    b_edge2 = b_edge.reshape(1, H)
    idx_lo = jnp.asarray(_IDX_LO, jnp.int32)
    idx_hi = jnp.asarray(_IDX_HI, jnp.int32)

    h0 = pl.pallas_call(
        _node_encode_body,
        grid=(N // _BN,),
        in_specs=[pl.BlockSpec((_BN, D_NODE), lambda i: (i, 0)),
                  _full((D_NODE, H)), _full((1, H))],
        out_specs=pl.BlockSpec((_BN, H), lambda i: (i, 0)),
        out_shape=jax.ShapeDtypeStruct((N, H), jnp.float32),
    )(x, W_node, b_node2)

    e_layers = []
    for l in range(L):
        e_l = pl.pallas_call(
            _edge_body,
            grid=(E // _BE,),
            in_specs=[pl.BlockSpec((_BE, D_EDGE), lambda i: (i, 0)),
                      _full((D_EDGE, H)), _full((1, H)),
                      _full((H, HW)), _full((H, HW)),
                      _full((1, HW)), _full((1, HW))],
            out_specs=pl.BlockSpec((_BE, HW), lambda i: (i, 0)),
            out_shape=jax.ShapeDtypeStruct((E, HW), jnp.int32),
        )(edge_attr, W_edge, b_edge2,
          W_elin[l][:, idx_lo], W_elin[l][:, idx_hi],
          b_elin[l][idx_lo].reshape(1, HW), b_elin[l][idx_hi].reshape(1, HW))
        e_layers.append(e_l)

    h = h0
    for l in range(L):
        parts = _sc_kernel(h, e_layers[l], src, dst)
        if l < L - 1:
            h = pl.pallas_call(
                _update_body,
                grid=(N // _BN,),
                in_specs=[pl.BlockSpec((_BN, H), lambda i: (i, 0)),
                          pl.BlockSpec((NC, _BN, H), lambda i: (0, i, 0)),
                          _full((H, H)), _full((1, H))],
                out_specs=pl.BlockSpec((_BN, H), lambda i: (i, 0)),
                out_shape=jax.ShapeDtypeStruct((N, H), jnp.float32),
            )(h, parts, W_nn[l], b_nn[l].reshape(1, H))
        else:
            out = pl.pallas_call(
                _final_body,
                grid=(N // _BN,),
                in_specs=[pl.BlockSpec((_BN, H), lambda i: (i, 0)),
                          pl.BlockSpec((NC, _BN, H), lambda i: (0, i, 0)),
                          pl.BlockSpec((_BN, H), lambda i: (i, 0)),
                          _full((H, H)), _full((1, H)),
                          _full((H, OUT)), _full((1, OUT)),
                          _full((1, OUT)), _full((1, OUT))],
                out_specs=pl.BlockSpec((_BN, OUT), lambda i: (i, 0)),
                out_shape=jax.ShapeDtypeStruct((N, OUT), jnp.float32),
            )(h, parts, h0, W_nn[l], b_nn[l].reshape(1, H),
              W_head, b_head.reshape(1, OUT),
              ln_g.reshape(1, OUT), ln_b.reshape(1, OUT))
    return out


# speedup vs baseline: 1.0396x; 1.0396x over previous
import functools

import jax
import jax.numpy as jnp
import numpy as np
from jax import lax
from jax.experimental import pallas as pl
from jax.experimental.pallas import tpu as pltpu
from jax.experimental.pallas import tpu_sc as plsc

_IDX_LO = np.concatenate([np.arange(16) + 32 * u for u in range(4)])
_IDX_HI = _IDX_LO + 16

N = 10000
E = 320000
D_NODE = 128
D_EDGE = 16
H = 128
OUT = 128
L = 3
HW = H // 2

NC = 2
NS = 16
NW = NC * NS
EPW = E // NW
CH = 40
NCHUNK = EPW // CH
NPAD = 10240
ROWS_PER_TILE = NPAD // NS
ZCHUNK = ROWS_PER_TILE // CH


def _sc_message_pass(h_hbm, e_hbm, src_hbm, dst_hbm, out_hbm,
                     table, sidx, didx,
                     hbuf0, hbuf1, ebuf0, ebuf1, mbuf,
                     sem_e0, sem_e1, sem_g0, sem_g1, sem_s,
                     sem_i0, sem_i1):
    c = lax.axis_index("c")
    s = lax.axis_index("s")
    wid = s * NC + c
    hb = (hbuf0, hbuf1)
    eb = (ebuf0, ebuf1)
    se = (sem_e0, sem_e1)
    sg = (sem_g0, sem_g1)
    si = (sem_i0, sem_i1)

    zero = jnp.zeros((16,), jnp.float32)

    def _zrow(i, carry):
        for g in range(8):
            mbuf[i, pl.ds(g * 16, 16)] = zero
        return carry

    lax.fori_loop(0, CH, _zrow, 0)
    for k in range(ZCHUNK):
        pltpu.sync_copy(mbuf, table.at[pl.ds(s * ROWS_PER_TILE + k * CH, CH)])
    plsc.subcore_barrier()

    def _issue_idx(k, p):
        base = wid * EPW + k * CH
        pltpu.async_copy(src_hbm.at[pl.ds(base, CH)], sidx.at[k % 8, 0], si[p])
        pltpu.async_copy(dst_hbm.at[pl.ds(base, CH)], didx.at[k % 8, 0], si[p])

    def _wait_idx(k, p):
        base = wid * EPW + k * CH
        pltpu.make_async_copy(src_hbm.at[pl.ds(base, CH)],
                              sidx.at[k % 8, 0], si[p]).wait()
        pltpu.make_async_copy(dst_hbm.at[pl.ds(base, CH)],
                              didx.at[k % 8, 0], si[p]).wait()

    def _issue_ge(k, b):
        pltpu.async_copy(e_hbm.at[pl.ds(wid * EPW + k * CH, CH)], eb[b], se[b])
        pltpu.async_copy(h_hbm.at[sidx.at[k % 8, 0]], hb[b], sg[b])

    def _wait_in(k, b):
        pltpu.make_async_copy(e_hbm.at[pl.ds(wid * EPW + k * CH, CH)],
                              eb[b], se[b]).wait()
        pltpu.make_async_copy(h_hbm.at[sidx.at[k % 8, 0]], hb[b], sg[b]).wait()

    hi_mask = jnp.int32(-65536)

    def _compute(b):
        hr, er, mr = hb[b], eb[b], mbuf

        def _row(i, cc):
            for u in range(4):
                we = er[i, pl.ds(u * 16, 16)]
                elo = lax.bitcast_convert_type(we << 16, jnp.float32)
                ehi = lax.bitcast_convert_type(we & hi_mask, jnp.float32)
                h0 = hr[i, pl.ds(u * 32, 16)]
                h1 = hr[i, pl.ds(u * 32 + 16, 16)]
                mr[i, pl.ds(u * 32, 16)] = jnp.maximum(h0 + elo, 0.0)
                mr[i, pl.ds(u * 32 + 16, 16)] = jnp.maximum(h1 + ehi, 0.0)
            return cc

        lax.fori_loop(0, CH, _row, 0)

    def _issue_sc(k):
        pltpu.async_copy(mbuf, table.at[didx.at[k % 8, 0]], sem_s, add=True)

    def _wait_sc(k):
        pltpu.make_async_copy(mbuf, table.at[didx.at[k % 8, 0]], sem_s).wait()

    def _step(k, b, wait_sc=True, ge2=True, idx4=True):
        _wait_in(k, b)
        if wait_sc:
            _wait_sc(k - 1)
        _compute(b)
        _issue_sc(k)
        if ge2:
            _wait_idx(k + 2, b)
            _issue_ge(k + 2, b)
        if idx4:
            _issue_idx(k + 4, b)

    _issue_idx(0, 0)
    _issue_idx(1, 1)
    _wait_idx(0, 0)
    _issue_ge(0, 0)
    _issue_idx(2, 0)
    _wait_idx(1, 1)
    _issue_ge(1, 1)
    _issue_idx(3, 1)
    _step(0, 0, wait_sc=False)
    _step(1, 1)

    def _pair(j, carry):
        for b in range(2):
            _step(2 * j + b, b)
        return carry

    _tail0 = 2 * ((NCHUNK - 4) // 2)
    lax.fori_loop(1, _tail0 // 2, _pair, 0)
    for k in range(_tail0, NCHUNK):
        _step(k, k % 2, ge2=(k + 2 < NCHUNK), idx4=(k + 4 < NCHUNK))
    _wait_sc(NCHUNK - 1)

    plsc.subcore_barrier()
    pltpu.sync_copy(table.at[pl.ds(s * ROWS_PER_TILE, ROWS_PER_TILE)],
                    out_hbm.at[c, pl.ds(s * ROWS_PER_TILE, ROWS_PER_TILE)])


@functools.cache
def _get_sc_kernel():
    return pl.kernel(
        _sc_message_pass,
        out_type=jax.ShapeDtypeStruct((NC, NPAD, H), jnp.float32),
        mesh=plsc.VectorSubcoreMesh(core_axis_name="c", subcore_axis_name="s",
                                    num_cores=NC, num_subcores=NS),
        scratch_types=[
            pltpu.VMEM_SHARED((NPAD, H), jnp.float32),
            pltpu.VMEM((8, 1, CH), jnp.int32),
            pltpu.VMEM((8, 1, CH), jnp.int32),
            pltpu.VMEM((CH, H), jnp.float32),
            pltpu.VMEM((CH, H), jnp.float32),
            pltpu.VMEM((CH, HW), jnp.int32),
            pltpu.VMEM((CH, HW), jnp.int32),
            pltpu.VMEM((CH, H), jnp.float32),
            pltpu.SemaphoreType.DMA,
            pltpu.SemaphoreType.DMA,
            pltpu.SemaphoreType.DMA,
            pltpu.SemaphoreType.DMA,
            pltpu.SemaphoreType.DMA,
            pltpu.SemaphoreType.DMA,
            pltpu.SemaphoreType.DMA,
        ],
    )


def _sc_kernel(h, e_l, src, dst):
    return _get_sc_kernel()(h, e_l, src, dst)


def _pack_words(lo, hi):
    lo_u = jax.lax.bitcast_convert_type(lo.astype(jnp.bfloat16),
                                        jnp.uint16).astype(jnp.uint32)
    hi_u = jax.lax.bitcast_convert_type(hi.astype(jnp.bfloat16),
                                        jnp.uint16).astype(jnp.uint32)
    return jax.lax.bitcast_convert_type(lo_u | (hi_u << 16), jnp.int32)


def _node_encode_body(x_ref, w_ref, b_ref, o_ref):
    o_ref[...] = jnp.dot(x_ref[...], w_ref[...],
                         preferred_element_type=jnp.float32) + b_ref[...]


def _edge_body(ea_ref, we_ref, be_ref, wl_ref, wh_ref, bl_ref, bh_ref,
               o0_ref, o1_ref, o2_ref):
    ea = ea_ref[...]
    we = we_ref[...]
    be = be_ref[...]
    outs = (o0_ref, o1_ref, o2_ref)
    for l in range(L):
        wl = wl_ref[l]
        wh = wh_ref[l]
        wcl = jnp.dot(we, wl, preferred_element_type=jnp.float32)
        wch = jnp.dot(we, wh, preferred_element_type=jnp.float32)
        bcl = jnp.dot(be, wl, preferred_element_type=jnp.float32) + bl_ref[l]
        bch = jnp.dot(be, wh, preferred_element_type=jnp.float32) + bh_ref[l]
        lo = jnp.dot(ea, wcl, preferred_element_type=jnp.float32) + bcl
        hi = jnp.dot(ea, wch, preferred_element_type=jnp.float32) + bch
        outs[l][...] = _pack_words(lo, hi)


def _update_body(h_ref, p_ref, w_ref, b_ref, o_ref):
    hp = h_ref[...] + p_ref[0] + p_ref[1]
    y = jnp.dot(hp, w_ref[...], preferred_element_type=jnp.float32) + b_ref[...]
    o_ref[...] = jnp.where(y >= 0, y, 0.01 * y)


def _final_body(h_ref, p_ref, skip_ref, wnn_ref, bnn_ref, wh_ref, bh_ref,
                g_ref, bb_ref, o_ref):
    hp = h_ref[...] + p_ref[0] + p_ref[1]
    y = jnp.dot(hp, wnn_ref[...], preferred_element_type=jnp.float32) + bnn_ref[...]
    h3 = jnp.where(y >= 0, y, 0.01 * y)
    hf = skip_ref[...] + h3
    out = jnp.dot(hf, wh_ref[...], preferred_element_type=jnp.float32) + bh_ref[...]
    mu = jnp.mean(out, axis=-1, keepdims=True)
    var = jnp.mean((out - mu) ** 2, axis=-1, keepdims=True)
    o_ref[...] = (out - mu) * lax.rsqrt(var + 1e-5) * g_ref[...] + bb_ref[...]


_BN = 2000
_BE = 4000


def _full(shape):
    return pl.BlockSpec(shape, lambda i: tuple(0 for _ in shape))


def kernel(x, edge_attr, edge_index, W_node, b_node, W_edge, b_edge,
           W_elin, b_elin, W_nn, b_nn, W_head, b_head, ln_g, ln_b):
    src = edge_index[0]
    dst = edge_index[1]
    b_node2 = b_node.reshape(1, H)
    b_edge2 = b_edge.reshape(1, H)
    idx_lo = jnp.asarray(_IDX_LO, jnp.int32)
    idx_hi = jnp.asarray(_IDX_HI, jnp.int32)

    h0 = pl.pallas_call(
        _node_encode_body,
        grid=(N // _BN,),
        in_specs=[pl.BlockSpec((_BN, D_NODE), lambda i: (i, 0)),
                  _full((D_NODE, H)), _full((1, H))],
        out_specs=pl.BlockSpec((_BN, H), lambda i: (i, 0)),
        out_shape=jax.ShapeDtypeStruct((N, H), jnp.float32),
    )(x, W_node, b_node2)

    e_spec = pl.BlockSpec((_BE, HW), lambda i: (i, 0))
    e_sds = jax.ShapeDtypeStruct((E, HW), jnp.int32)
    e_layers = pl.pallas_call(
        _edge_body,
        grid=(E // _BE,),
        in_specs=[pl.BlockSpec((_BE, D_EDGE), lambda i: (i, 0)),
                  _full((D_EDGE, H)), _full((1, H)),
                  _full((L, H, HW)), _full((L, H, HW)),
                  _full((L, 1, HW)), _full((L, 1, HW))],
        out_specs=[e_spec, e_spec, e_spec],
        out_shape=[e_sds, e_sds, e_sds],
    )(edge_attr, W_edge, b_edge2,
      W_elin[:, :, idx_lo], W_elin[:, :, idx_hi],
      b_elin[:, idx_lo].reshape(L, 1, HW), b_elin[:, idx_hi].reshape(L, 1, HW))

    h = h0
    for l in range(L):
        parts = _sc_kernel(h, e_layers[l], src, dst)
        if l < L - 1:
            h = pl.pallas_call(
                _update_body,
                grid=(N // _BN,),
                in_specs=[pl.BlockSpec((_BN, H), lambda i: (i, 0)),
                          pl.BlockSpec((NC, _BN, H), lambda i: (0, i, 0)),
                          _full((H, H)), _full((1, H))],
                out_specs=pl.BlockSpec((_BN, H), lambda i: (i, 0)),
                out_shape=jax.ShapeDtypeStruct((N, H), jnp.float32),
            )(h, parts, W_nn[l], b_nn[l].reshape(1, H))
        else:
            out = pl.pallas_call(
                _final_body,
                grid=(N // _BN,),
                in_specs=[pl.BlockSpec((_BN, H), lambda i: (i, 0)),
                          pl.BlockSpec((NC, _BN, H), lambda i: (0, i, 0)),
                          pl.BlockSpec((_BN, H), lambda i: (i, 0)),
                          _full((H, H)), _full((1, H)),
                          _full((H, OUT)), _full((1, OUT)),
                          _full((1, OUT)), _full((1, OUT))],
                out_specs=pl.BlockSpec((_BN, OUT), lambda i: (i, 0)),
                out_shape=jax.ShapeDtypeStruct((N, OUT), jnp.float32),
            )(h, parts, h0, W_nn[l], b_nn[l].reshape(1, H),
              W_head, b_head.reshape(1, OUT),
              ln_g.reshape(1, OUT), ln_b.reshape(1, OUT))
    return out
